# Initial kernel scaffold; baseline (speedup 1.0000x reference)
import jax
import jax.numpy as jnp
from jax.experimental import pallas as pl


def kernel(x, edge_index, W, b):
    raise NotImplementedError("write your pallas kernel here")


# speedup vs baseline: 3.3732x; 3.3732x over previous
import jax
import jax.numpy as jnp
from jax import lax
from jax.experimental import pallas as pl
from jax.experimental.pallas import tpu as pltpu
from jax.experimental.pallas import tpu_sc as plsc

N = 10000
DIN = 256
DH = 128
E = 160000
NSUB = 16
NCORE = 2
CHUNK = 128
NCHUNK = 80
EPAD = NSUB * NCHUNK * CHUNK
NPAD = 10112
RPS = NPAD // NSUB
CW = 16


def _sc_body(x0, x1, srcs, dsts, zc, ones_h, agg_out, cnt_out,
             agg_sh, cnt_sh, src_v, dst_v, gbuf, ones_v, sem):
    c = lax.axis_index("c")
    s = lax.axis_index("s")
    rows = pl.ds(s * RPS, RPS)

    pltpu.sync_copy(srcs.at[s], src_v)
    pltpu.sync_copy(dsts.at[s], dst_v)

    @pl.when(c == 0)
    def _():
        pltpu.sync_copy(x0.at[rows], agg_sh.at[rows])
        pltpu.sync_copy(zc.at[rows], cnt_sh.at[rows])
        pltpu.sync_copy(ones_h, ones_v)

    @pl.when(c == 1)
    def _():
        pltpu.sync_copy(x1.at[rows], agg_sh.at[rows])

    plsc.subcore_barrier()

    def chunk0(j, carry):
        pltpu.async_copy(x0.at[src_v.at[j]], gbuf, sem).wait()
        pltpu.sync_copy(gbuf, agg_sh.at[dst_v.at[j]], add=True)
        pltpu.sync_copy(ones_v, cnt_sh.at[dst_v.at[j]], add=True)
        return carry

    def chunk1(j, carry):
        pltpu.async_copy(x1.at[src_v.at[j]], gbuf, sem).wait()
        pltpu.sync_copy(gbuf, agg_sh.at[dst_v.at[j]], add=True)
        return carry

    @pl.when(c == 0)
    def _():
        lax.fori_loop(0, NCHUNK, chunk0, 0)

    @pl.when(c == 1)
    def _():
        lax.fori_loop(0, NCHUNK, chunk1, 0)

    plsc.subcore_barrier()

    pltpu.sync_copy(agg_sh.at[rows], agg_out.at[c, rows])

    @pl.when(c == 0)
    def _():
        pltpu.sync_copy(cnt_sh.at[rows], cnt_out.at[rows])


def _sc_aggregate(x0, x1, srcs, dsts, zc, ones_h):
    mesh = plsc.VectorSubcoreMesh(core_axis_name="c", subcore_axis_name="s",
                                  num_cores=NCORE, num_subcores=NSUB)
    return pl.kernel(
        _sc_body,
        out_type=(jax.ShapeDtypeStruct((NCORE, NPAD, DH), jnp.float32),
                  jax.ShapeDtypeStruct((NPAD, CW), jnp.float32)),
        mesh=mesh,
        scratch_types=[
            pltpu.VMEM_SHARED((NPAD, DH), jnp.float32),
            pltpu.VMEM_SHARED((NPAD, CW), jnp.float32),
            pltpu.VMEM((NCHUNK, CHUNK), jnp.int32),
            pltpu.VMEM((NCHUNK, CHUNK), jnp.int32),
            pltpu.VMEM((CHUNK, DH), jnp.float32),
            pltpu.VMEM((CHUNK, CW), jnp.float32),
            pltpu.SemaphoreType.DMA,
        ],
        compiler_params=pltpu.CompilerParams(use_tc_tiling_on_sc=False),
    )(x0, x1, srcs, dsts, zc, ones_h)


def _tc_body(agg_ref, cnt_ref, w_ref, b_ref, out_ref):
    a = jnp.concatenate([agg_ref[0], agg_ref[1]], axis=1)
    denom = 1.0 + jnp.maximum(cnt_ref[:, 0:1], 1.0)
    a = a / denom
    o = lax.dot_general(a, w_ref[:], (((1,), (1,)), ((), ())),
                        preferred_element_type=jnp.float32)
    o = o + b_ref[:]
    out_ref[:] = jnp.where(o >= 0.0, o, 0.2 * o)


def _tc_tail(agg, cnt, w, b2):
    bm = 512
    return pl.pallas_call(
        _tc_body,
        grid=(pl.cdiv(N, bm),),
        in_specs=[
            pl.BlockSpec((NCORE, bm, DH), lambda i: (0, i, 0)),
            pl.BlockSpec((bm, CW), lambda i: (i, 0)),
            pl.BlockSpec((DIN, DIN), lambda i: (0, 0)),
            pl.BlockSpec((1, DIN), lambda i: (0, 0)),
        ],
        out_specs=pl.BlockSpec((bm, DIN), lambda i: (i, 0)),
        out_shape=jax.ShapeDtypeStruct((N, DIN), jnp.float32),
    )(agg, cnt, w, b2)


def kernel(x, edge_index, W, b):
    src = edge_index[0].astype(jnp.int32)
    dst = edge_index[1].astype(jnp.int32)
    pad = EPAD - E
    src = jnp.concatenate([src, jnp.zeros((pad,), jnp.int32)])
    dst = jnp.concatenate(
        [dst, N + (jnp.arange(pad, dtype=jnp.int32) % (NPAD - N))])
    srcs = src.reshape(NSUB, NCHUNK, CHUNK)
    dsts = dst.reshape(NSUB, NCHUNK, CHUNK)
    xp = jnp.pad(x, ((0, NPAD - N), (0, 0)))
    x0 = xp[:, :DH]
    x1 = xp[:, DH:]
    zc = jnp.zeros((NPAD, CW), jnp.float32)
    ones_h = jnp.ones((CHUNK, CW), jnp.float32)
    agg, cnt = _sc_aggregate(x0, x1, srcs, dsts, zc, ones_h)
    return _tc_tail(agg, cnt, W, b.reshape(1, DIN))
